# baseline (speedup 1.0000x reference)
import functools
import numpy as np
import jax
import jax.numpy as jnp
from jax import lax
from jax.experimental import pallas as pl
from jax.experimental.pallas import tpu as pltpu
from jax.experimental.pallas import tpu_sc as plsc

_LAYERS = [[1, 0, 20, 26, 25, 45],
           [0, 20, 12, 16, 2, 4, 8, 25, 45, 37, 41, 27, 29, 33],
           [12, 16, 2, 4, 8, 13, 17, 3, 5, 9, 3, 28, 37, 41, 27, 29, 33, 38, 42, 28, 30, 34],
           [13, 17, 3, 5, 9, 14, 18, 6, 10, 3, 28, 38, 42, 28, 30, 34, 39, 43, 31, 35],
           [14, 18, 6, 10, 15, 19, 7, 11, 39, 43, 31, 35, 40, 44, 32, 36],
           [15, 19, 7, 11, 21, 22, 23, 24, 40, 44, 32, 36, 46, 47, 48, 49]]

_S_np = np.zeros((6, 50), np.float32)
for _i, _lst in enumerate(_LAYERS):
    for _v in _lst:
        _S_np[_i, _v] += 1.0
_SE_np = np.tile(_S_np, (16, 1))
_LEN_np = np.tile(np.array([[len(l)] for l in _LAYERS], np.float32), (16, 1))

_NEG = -1e30


def _max_body(x_ref, o_ref):
    o_ref[...] = jnp.max(x_ref[...], axis=1)


def _pre_body(xt_ref, SE_ref, LEN_ref, WdT_ref, bd_ref, g1_ref, b1_ref,
              xs_ref, pd_ref):
    R, C = xt_ref.shape
    IC = WdT_ref.shape[1]
    NL = SE_ref.shape[0]
    V = R // NL
    Nn = 16
    L = NL // Nn
    def _bf16_round(v):
        u = jax.lax.bitcast_convert_type(v, jnp.int32)
        r = (u + 0x7FFF + ((u >> 16) & 1)) & ~0xFFFF
        return jax.lax.bitcast_convert_type(r, jnp.float32)

    h = jnp.dot(_bf16_round(xt_ref[...]), _bf16_round(WdT_ref[...]),
                preferred_element_type=jnp.float32)
    h = h + bd_ref[...]
    mu = jnp.mean(h, axis=0, keepdims=True)
    var = jnp.mean((h - mu) ** 2, axis=0, keepdims=True)
    h = (h - mu) / jnp.sqrt(var + 1e-5)
    h = h * g1_ref[...] + b1_ref[...]
    h = jnp.maximum(h, 0.0)
    h3 = h.reshape(NL, V, IC)
    xs = jnp.sum(h3 * SE_ref[...][:, :, None], axis=1) / LEN_ref[...]
    xs_ref[...] = xs
    xs3 = xs.reshape(Nn, L, IC)
    prod = xs3[:, :, None, :] * xs3[:, None, :, :]
    raw = jnp.sum(prod, axis=3)
    xx = jnp.sum(xs3 * xs3, axis=2)
    pd = 2.0 * raw - xx[:, :, None] - xx[:, None, :]
    pd_ref[...] = jnp.concatenate(
        [pd.reshape(NL, L), jnp.full((NL, 16 - L), _NEG, jnp.float32)],
        axis=1)


def _sc_knn_body(pd_hbm, xs_hbm, out_hbm, pd_v, xs_v, ctr_v, out_v):
    wid = lax.axis_index("s") * 2 + lax.axis_index("c")
    base = wid * 8

    @pl.when(wid < 12)
    def _work():
        pltpu.sync_copy(pd_hbm.at[pl.ds(base, 8)], pd_v)
        pltpu.sync_copy(xs_hbm, xs_v)
        pltpu.sync_copy(xs_hbm.at[pl.ds(base, 8)], ctr_v)
        lanes = lax.broadcasted_iota(jnp.int32, (16,), 0)
        for i in range(8):
            grp = ((base + i) // 6) * 6
            row = pd_v[i, :]
            for k in range(3):
                mx = jnp.max(row)
                idxv = plsc.all_reduce_ffs(row == mx)
                rowv = jnp.broadcast_to(idxv + grp, (16,)).astype(jnp.int32)
                for c in range(4):
                    col = lanes + (16 * c)
                    nb = plsc.load_gather(xs_v, [rowv, col])
                    ctr = ctr_v[i, pl.ds(16 * c, 16)]
                    out_v[k, i, pl.ds(16 * c, 16)] = nb - ctr
                row = jnp.where(lanes == idxv, _NEG, row)
        for k in range(3):
            pltpu.sync_copy(out_v.at[k], out_hbm.at[k, pl.ds(base, 8)])


def _post_body(xs_ref, df_ref, AT_ref, BT_ref, g2_ref, b2_ref,
               WaggT_ref, bagg_ref, out_ref):
    NL, IC = xs_ref.shape
    xs = xs_ref[...]
    base = jnp.dot(xs, BT_ref[...], preferred_element_type=jnp.float32)
    df = df_ref[...].reshape(3, NL, IC)
    es = [jnp.dot(df[k], AT_ref[...], preferred_element_type=jnp.float32)
          + base for k in range(3)]
    e = jnp.stack(es, axis=0)
    mu2 = jnp.mean(e, axis=(0, 1), keepdims=True)
    var2 = jnp.mean((e - mu2) ** 2, axis=(0, 1), keepdims=True)
    e = (e - mu2) / jnp.sqrt(var2 + 1e-5)
    e = e * g2_ref[...] + b2_ref[...]
    e = jnp.where(e > 0, e, 0.2 * e)
    att0 = jnp.max(e, axis=0)
    att = jnp.dot(att0, WaggT_ref[...], preferred_element_type=jnp.float32)
    att = att + bagg_ref[...]
    out_ref[...] = jax.nn.sigmoid(att)


def _gate_body(x_ref, g_ref, o_ref):
    acc = x_ref[0, 0] * g_ref[0, 0][None, :]
    for l in range(1, x_ref.shape[1]):
        acc = acc + x_ref[0, l] * g_ref[0, l][None, :]
    o_ref[0] = acc


def kernel(x, W_down, b_down, gamma1, beta1, W_ec, gamma2, beta2, W_agg, b_agg):
    N, C, L, T, V = x.shape
    IC = W_down.shape[0]
    y = jnp.transpose(x, (0, 2, 4, 3, 1))
    RB = 96
    xt = pl.pallas_call(
        _max_body,
        grid=(N * L * V // RB,),
        in_specs=[pl.BlockSpec((RB, T, C), lambda r: (r, 0, 0))],
        out_specs=pl.BlockSpec((RB, C), lambda r: (r, 0)),
        out_shape=jax.ShapeDtypeStruct((N * L * V, C), x.dtype),
    )(y.reshape(N * L * V, T, C))
    xs, pd = pl.pallas_call(
        _pre_body,
        out_shape=(jax.ShapeDtypeStruct((N * L, IC), jnp.float32),
                   jax.ShapeDtypeStruct((N * L, 16), jnp.float32)),
    )(xt, jnp.asarray(_SE_np), jnp.asarray(_LEN_np), W_down.T, b_down,
      gamma1, beta1)
    mesh = plsc.VectorSubcoreMesh(core_axis_name="c", subcore_axis_name="s")
    knn = functools.partial(
        pl.kernel,
        mesh=mesh,
        compiler_params=pltpu.CompilerParams(needs_layout_passes=False),
        out_type=jax.ShapeDtypeStruct((3, N * L, IC), jnp.float32),
        scratch_types=[pltpu.VMEM((8, 16), jnp.float32),
                       pltpu.VMEM((N * L, IC), jnp.float32),
                       pltpu.VMEM((8, IC), jnp.float32),
                       pltpu.VMEM((3, 8, IC), jnp.float32)],
    )(_sc_knn_body)
    diffs = knn(pd, xs).reshape(3 * N * L, IC)
    gates = pl.pallas_call(
        _post_body,
        out_shape=jax.ShapeDtypeStruct((N * L, C), jnp.float32),
    )(xs, diffs, W_ec[:, :IC].T, W_ec[:, IC:].T, gamma2, beta2,
      W_agg.T, b_agg)
    MB = 1600
    out_y = pl.pallas_call(
        _gate_body,
        grid=(N, V * T // MB),
        in_specs=[pl.BlockSpec((1, L, MB, C), lambda n, mb: (n, 0, mb, 0)),
                  pl.BlockSpec((1, L, C), lambda n, mb: (n, 0, 0))],
        out_specs=pl.BlockSpec((1, MB, C), lambda n, mb: (n, mb, 0)),
        out_shape=jax.ShapeDtypeStruct((N, V * T, C), x.dtype),
    )(y.reshape(N, L, V * T, C), gates.reshape(N, L, C))
    return jnp.transpose(out_y.reshape(N, V, T, C), (0, 3, 2, 1))
